# Initial kernel scaffold; baseline (speedup 1.0000x reference)
import jax
import jax.numpy as jnp
from jax.experimental import pallas as pl


def kernel(x, edge_index, batch, W1, b1, gamma1, beta1, Wc, bc, Wout, bout, Wg1, bg1, Wg2, bg2):
    raise NotImplementedError("write your pallas kernel here")


# speedup vs baseline: 11.7626x; 11.7626x over previous
import functools

import jax
import jax.numpy as jnp
from jax import lax
from jax.experimental import pallas as pl
from jax.experimental.pallas import tpu as pltpu
from jax.experimental.pallas import tpu_sc as plsc

N = 10000
E = 320000
NUM_SEG = 10000
D = 128
NUM_GRAPHS = 8
EPS_BN = 1e-5

NC = 2
NS = 16
NW = NC * NS
E_PER_W = E // NW
CHUNK = 80
NCHUNK = E_PER_W // CHUNK
SEG_PAD = 10240
ROWS_PER_TILE = SEG_PAD // NS
ZROWS = 128
CPT = SEG_PAD // NS


def _sc_degrees_body(nidx_hbm, hidx_hbm, out_hbm,
                     locn, loch, ibufn, ibufh, sp, red, res):
    c = lax.axis_index("c")
    s = lax.axis_index("s")
    wid = c * NS + s
    base = pl.multiple_of(wid * E_PER_W, 8)

    zero16 = jnp.zeros((16,), jnp.float32)
    one16 = jnp.ones((16,), jnp.float32)

    def zz(i, _):
        locn[pl.ds(i * 16, 16)] = zero16
        loch[pl.ds(i * 16, 16)] = zero16
        return 0
    lax.fori_loop(0, SEG_PAD // 16, zz, 0)

    pltpu.sync_copy(nidx_hbm.at[pl.ds(base, E_PER_W)], ibufn)
    pltpu.sync_copy(hidx_hbm.at[pl.ds(base, E_PER_W)], ibufh)

    def cnt(k, _):
        plsc.addupdate_scatter(locn, [ibufn[pl.ds(k * 16, 16)]], one16)
        plsc.addupdate_scatter(loch, [ibufh[pl.ds(k * 16, 16)]], one16)
        return 0
    lax.fori_loop(0, E_PER_W // 16, cnt, 0)

    pltpu.sync_copy(locn, sp.at[0, s])
    pltpu.sync_copy(loch, sp.at[1, s])
    plsc.subcore_barrier()
    colbase = pl.multiple_of(s * CPT, 128)
    for tbl in range(2):
        pltpu.sync_copy(sp.at[tbl, :, pl.ds(colbase, CPT)], red)

        def rq(q, _):
            acc = red[0, pl.ds(q * 16, 16)]
            for r in range(1, NS):
                acc = acc + red[r, pl.ds(q * 16, 16)]
            res[pl.ds(q * 16, 16)] = acc
            return 0
        lax.fori_loop(0, CPT // 16, rq, 0)
        pltpu.sync_copy(res, out_hbm.at[c, tbl, pl.ds(colbase, CPT)])


def _sc_degrees(nidx, hidx):
    mesh = plsc.VectorSubcoreMesh(core_axis_name="c", subcore_axis_name="s")
    kern = functools.partial(
        pl.kernel,
        mesh=mesh,
        compiler_params=pltpu.CompilerParams(needs_layout_passes=False),
        out_type=jax.ShapeDtypeStruct((NC, 2, SEG_PAD), jnp.float32),
        scratch_types=[
            pltpu.VMEM((SEG_PAD,), jnp.float32),
            pltpu.VMEM((SEG_PAD,), jnp.float32),
            pltpu.VMEM((E_PER_W,), jnp.int32),
            pltpu.VMEM((E_PER_W,), jnp.int32),
            pltpu.VMEM_SHARED((2, NS, SEG_PAD), jnp.float32),
            pltpu.VMEM((NS, CPT), jnp.float32),
            pltpu.VMEM((CPT,), jnp.float32),
        ],
    )(_sc_degrees_body)
    return kern(nidx, hidx)


def _sc_propagate_body(table, gidx_hbm, sidx_hbm, out_hbm,
                       acc, idx_g, idx_s, rows, zbuf, sem):
    c = lax.axis_index("c")
    s = lax.axis_index("s")
    wid = c * NS + s

    zero16 = jnp.zeros((16,), jnp.float32)

    def zrow(i, _):
        r = i // (D // 16)
        k = i % (D // 16)
        zbuf[r, pl.ds(k * 16, 16)] = zero16
        return 0
    lax.fori_loop(0, ZROWS * (D // 16), zrow, 0)

    def zcp(q, _):
        ro = pl.multiple_of(s * ROWS_PER_TILE + q * ZROWS, 8)
        pltpu.sync_copy(zbuf, acc.at[pl.ds(ro, ZROWS)])
        return 0
    lax.fori_loop(0, ROWS_PER_TILE // ZROWS, zcp, 0)
    plsc.subcore_barrier()

    base = wid * E_PER_W

    def step(j, _):
        off = pl.multiple_of(base + j * CHUNK, 8)
        pltpu.sync_copy(gidx_hbm.at[pl.ds(off, CHUNK)], idx_g)
        pltpu.sync_copy(sidx_hbm.at[pl.ds(off, CHUNK)], idx_s)
        pltpu.async_copy(table.at[idx_g], rows, sem).wait()
        pltpu.sync_copy(rows, acc.at[idx_s], add=True)
        return 0
    lax.fori_loop(0, NCHUNK, step, 0)
    plsc.subcore_barrier()

    ro = pl.multiple_of(s * ROWS_PER_TILE, 8)
    pltpu.sync_copy(acc.at[pl.ds(ro, ROWS_PER_TILE)],
                    out_hbm.at[c, pl.ds(ro, ROWS_PER_TILE)])


def _sc_propagate(table, gidx, sidx):
    mesh = plsc.VectorSubcoreMesh(core_axis_name="c", subcore_axis_name="s")
    kern = functools.partial(
        pl.kernel,
        mesh=mesh,
        compiler_params=pltpu.CompilerParams(needs_layout_passes=False),
        out_type=jax.ShapeDtypeStruct((NC, SEG_PAD, D), jnp.float32),
        scratch_types=[
            pltpu.VMEM_SHARED((SEG_PAD, D), jnp.float32),
            pltpu.VMEM((CHUNK,), jnp.int32),
            pltpu.VMEM((CHUNK,), jnp.int32),
            pltpu.VMEM((CHUNK, D), jnp.float32),
            pltpu.VMEM((ZROWS, D), jnp.float32),
            pltpu.SemaphoreType.DMA,
        ],
    )(_sc_propagate_body)
    return kern(table, gidx, sidx)


def _stage1_body(x, W1, b1s, Wc, out):
    h = jnp.dot(x[...], W1[...], preferred_element_type=jnp.float32)
    h = jax.nn.relu(h + b1s[...])
    out[...] = jnp.dot(h, Wc[...], preferred_element_type=jnp.float32)


def _stage1(x, W1, b1s, Wc):
    blk = 2000
    return pl.pallas_call(
        _stage1_body,
        grid=(N // blk,),
        in_specs=[
            pl.BlockSpec((blk, D), lambda i: (i, 0)),
            pl.BlockSpec((D, D), lambda i: (0, 0)),
            pl.BlockSpec((1, D), lambda i: (0, 0)),
            pl.BlockSpec((D, D), lambda i: (0, 0)),
        ],
        out_specs=pl.BlockSpec((blk, D), lambda i: (i, 0)),
        out_shape=jax.ShapeDtypeStruct((N, D), jnp.float32),
    )(x, W1, b1s, Wc)


def _stage2_body(p, cnts, out):
    s = p[0] + p[1]
    cnt = (cnts[0, 1] + cnts[1, 1]).reshape(-1, 1)
    inv = jnp.where(cnt > 0, 1.0 / cnt, 0.0)
    out[...] = s * inv


def _stage2(partials, counts):
    blk = 2048
    return pl.pallas_call(
        _stage2_body,
        grid=(SEG_PAD // blk,),
        in_specs=[
            pl.BlockSpec((NC, blk, D), lambda i: (0, i, 0)),
            pl.BlockSpec((NC, 2, blk), lambda i: (0, 0, i)),
        ],
        out_specs=pl.BlockSpec((blk, D), lambda i: (i, 0)),
        out_shape=jax.ShapeDtypeStruct((SEG_PAD, D), jnp.float32),
    )(partials, counts)


def _stage3_body(p, cnts, batch, bc, Wout, bout, Wg1, bg1, Wg2, bg2, out):
    s = p[0, :N, :] + p[1, :N, :]
    cnt = (cnts[0, 0, :N] + cnts[1, 0, :N]).reshape(N, 1)
    inv = jnp.where(cnt > 0, 1.0 / cnt, 0.0)
    h = s * inv + bc[...]
    o = jnp.dot(h, Wout[...], preferred_element_type=jnp.float32) + bout[...]
    g1 = jnp.tanh(
        jnp.dot(o, Wg1[...], preferred_element_type=jnp.float32) + bg1[...])
    gate = jnp.dot(g1, Wg2[...], preferred_element_type=jnp.float32) + bg2[...]
    b = batch[...]
    gid = lax.broadcasted_iota(jnp.int32, (1, NUM_GRAPHS), 1)
    mask = b == gid
    maskf = mask.astype(jnp.float32)
    gmax = jnp.max(jnp.where(mask, gate, -1e30), axis=0, keepdims=True)
    grow = jnp.sum(maskf * gmax, axis=1, keepdims=True)
    e = jnp.exp(gate - grow)
    denom = jnp.sum(maskf * e, axis=0, keepdims=True)
    drow = jnp.sum(maskf * denom, axis=1, keepdims=True)
    alpha = e / (drow + 1e-16)
    w = maskf * alpha
    out[...] = lax.dot_general(
        w, o, dimension_numbers=(((0,), (0,)), ((), ())),
        preferred_element_type=jnp.float32)


def _stage3(partials, counts, batch2d, bc, Wout, bout, Wg1, bg1, Wg2, bg2):
    return pl.pallas_call(
        _stage3_body,
        out_shape=jax.ShapeDtypeStruct((NUM_GRAPHS, D), jnp.float32),
    )(partials, counts, batch2d, bc, Wout, bout, Wg1, bg1, Wg2, bg2)


def kernel(x, edge_index, batch, W1, b1, gamma1, beta1, Wc, bc, Wout, bout,
           Wg1, bg1, Wg2, bg2):
    node_idx = edge_index[0]
    he_idx = edge_index[1]
    g = gamma1 / jnp.sqrt(1.0 + EPS_BN)
    W1f = W1 * g[None, :]
    b1f = (b1 * g + beta1).reshape(1, D)

    counts = _sc_degrees(node_idx, he_idx)
    h2 = _stage1(x, W1f, b1f, Wc)
    p1 = _sc_propagate(h2, node_idx, he_idx)
    ef = _stage2(p1, counts)
    p2 = _sc_propagate(ef, he_idx, node_idx)
    return _stage3(p2, counts, batch.reshape(N, 1), bc.reshape(1, D),
                   Wout, bout.reshape(1, D), Wg1, bg1.reshape(1, D // 2),
                   Wg2, bg2.reshape(1, 1))


# speedup vs baseline: 19.1663x; 1.6294x over previous
import functools

import jax
import jax.numpy as jnp
from jax import lax
from jax.experimental import pallas as pl
from jax.experimental.pallas import tpu as pltpu
from jax.experimental.pallas import tpu_sc as plsc

N = 10000
E = 320000
NUM_SEG = 10000
D = 128
NUM_GRAPHS = 8
EPS_BN = 1e-5

NC = 2
NS = 16
NW = NC * NS
E_PER_W = E // NW
CHUNK = 40
PASSES = 2
E_PER_P = E_PER_W // PASSES
NCHUNK_P = E_PER_P // CHUNK
NCP_PAD = 128
IG = 16
GSTAGE = 1000
SEG_PAD = 10240
ROWS_PER_TILE = SEG_PAD // NS
ZROWS = 40
CPT = SEG_PAD // NS


def _sc_degrees_body(nidx_hbm, hidx_hbm, out_hbm,
                     locn, loch, ibufn, ibufh):
    c = lax.axis_index("c")
    s = lax.axis_index("s")
    wid = c * NS + s
    base = pl.multiple_of(wid * E_PER_W, 8)

    zero16 = jnp.zeros((16,), jnp.float32)
    one16 = jnp.ones((16,), jnp.float32)

    def zz(i, _):
        locn[pl.ds(i * 16, 16)] = zero16
        loch[pl.ds(i * 16, 16)] = zero16
        return 0
    lax.fori_loop(0, SEG_PAD // 16, zz, 0)

    pltpu.sync_copy(nidx_hbm.at[pl.ds(base, E_PER_W)], ibufn)
    pltpu.sync_copy(hidx_hbm.at[pl.ds(base, E_PER_W)], ibufh)

    def cnt(k, _):
        plsc.addupdate_scatter(locn, [ibufn[pl.ds(k * 16, 16)]], one16)
        plsc.addupdate_scatter(loch, [ibufh[pl.ds(k * 16, 16)]], one16)
        return 0
    lax.fori_loop(0, E_PER_W // 16, cnt, 0)

    pltpu.sync_copy(locn, out_hbm.at[wid, 0])
    pltpu.sync_copy(loch, out_hbm.at[wid, 1])


def _sc_degrees(nidx, hidx):
    mesh = plsc.VectorSubcoreMesh(core_axis_name="c", subcore_axis_name="s")
    kern = functools.partial(
        pl.kernel,
        mesh=mesh,
        compiler_params=pltpu.CompilerParams(needs_layout_passes=False),
        out_type=jax.ShapeDtypeStruct((NW, 2, SEG_PAD), jnp.float32),
        scratch_types=[
            pltpu.VMEM((SEG_PAD,), jnp.float32),
            pltpu.VMEM((SEG_PAD,), jnp.float32),
            pltpu.VMEM((E_PER_W,), jnp.int32),
            pltpu.VMEM((E_PER_W,), jnp.int32),
        ],
    )(_sc_degrees_body)
    return kern(nidx, hidx)


def _sc_propagate_body(table, gidx_hbm, sidx_hbm, out_hbm,
                       acc, gbuf, sbuf, rows, zbuf, semg):
    c = lax.axis_index("c")
    s = lax.axis_index("s")
    wid = c * NS + s

    zero16 = jnp.zeros((16,), jnp.float32)

    def zrow(i, _):
        r = i // (D // 16)
        k = i % (D // 16)
        zbuf[r, pl.ds(k * 16, 16)] = zero16
        return 0
    lax.fori_loop(0, ZROWS * (D // 16), zrow, 0)

    def zcp(q, _):
        ro = pl.multiple_of(s * ROWS_PER_TILE + q * ZROWS, 8)
        pltpu.sync_copy(zbuf, acc.at[pl.ds(ro, ZROWS)])
        return 0
    lax.fori_loop(0, ROWS_PER_TILE // ZROWS, zcp, 0)
    plsc.subcore_barrier()

    def run_pass(p, _):
        def gcp(g, _):
            ro = pl.multiple_of(g * GSTAGE, 8)
            src = pl.multiple_of(wid * E_PER_W + p * E_PER_P + ro, 8)
            pltpu.sync_copy(gidx_hbm.at[pl.ds(src, GSTAGE)],
                            gbuf.at[pl.ds(ro, GSTAGE)])
            return 0
        lax.fori_loop(0, E_PER_P // GSTAGE, gcp, 0)

        def scp(g, _):
            ro = pl.multiple_of(g * IG, 8)
            pltpu.sync_copy(sidx_hbm.at[wid, p, pl.ds(ro, IG)],
                            sbuf.at[pl.ds(ro, IG)])
            return 0
        lax.fori_loop(0, NCP_PAD // IG, scp, 0)

        def rslice(j):
            off = pl.multiple_of((j % 2) * CHUNK, 8)
            return rows.at[pl.ds(off, CHUNK)]

        def gslice(j):
            off = pl.multiple_of(j * CHUNK, 8)
            return gbuf.at[pl.ds(off, CHUNK)]

        def step(t, _):
            @pl.when(t < NCHUNK_P)
            def _():
                pltpu.async_copy(table.at[gslice(t)], rslice(t), semg)

            @pl.when(t >= 1)
            def _():
                j = t - 1
                pltpu.make_async_copy(table.at[gslice(j)], rslice(j),
                                      semg).wait()
                pltpu.sync_copy(rslice(j), acc.at[sbuf.at[j]], add=True)
            return 0
        lax.fori_loop(0, NCHUNK_P + 1, step, 0)
        return 0

    lax.fori_loop(0, PASSES, run_pass, 0)
    plsc.subcore_barrier()

    ro = pl.multiple_of(s * ROWS_PER_TILE, 8)
    pltpu.sync_copy(acc.at[pl.ds(ro, ROWS_PER_TILE)],
                    out_hbm.at[c, pl.ds(ro, ROWS_PER_TILE)])


def _sc_propagate(table, gidx, sidx):
    mesh = plsc.VectorSubcoreMesh(core_axis_name="c", subcore_axis_name="s")
    kern = functools.partial(
        pl.kernel,
        mesh=mesh,
        compiler_params=pltpu.CompilerParams(needs_layout_passes=False),
        out_type=jax.ShapeDtypeStruct((NC, SEG_PAD, D), jnp.float32),
        scratch_types=[
            pltpu.VMEM_SHARED((SEG_PAD, D), jnp.float32),
            pltpu.VMEM((E_PER_P,), jnp.int32),
            pltpu.VMEM((NCP_PAD, CHUNK), jnp.int32),
            pltpu.VMEM((2 * CHUNK, D), jnp.float32),
            pltpu.VMEM((ZROWS, D), jnp.float32),
            pltpu.SemaphoreType.DMA,
        ],
    )(_sc_propagate_body)
    sidx_p = jnp.pad(sidx.reshape(NW, PASSES, NCHUNK_P, CHUNK),
                     ((0, 0), (0, 0), (0, NCP_PAD - NCHUNK_P), (0, 0)))
    return kern(table, gidx, sidx_p)


def _stage1_body(x, W1, b1s, Wc, out):
    h = jnp.dot(x[...], W1[...], preferred_element_type=jnp.float32)
    h = jax.nn.relu(h + b1s[...])
    out[...] = jnp.dot(h, Wc[...], preferred_element_type=jnp.float32)


def _stage1(x, W1, b1s, Wc):
    blk = 2000
    return pl.pallas_call(
        _stage1_body,
        grid=(N // blk,),
        in_specs=[
            pl.BlockSpec((blk, D), lambda i: (i, 0)),
            pl.BlockSpec((D, D), lambda i: (0, 0)),
            pl.BlockSpec((1, D), lambda i: (0, 0)),
            pl.BlockSpec((D, D), lambda i: (0, 0)),
        ],
        out_specs=pl.BlockSpec((blk, D), lambda i: (i, 0)),
        out_shape=jax.ShapeDtypeStruct((N, D), jnp.float32),
    )(x, W1, b1s, Wc)


def _stage2_body(p, cnts, out):
    s = p[0] + p[1]
    cnt = jnp.sum(cnts[:, 1, :], axis=0).reshape(-1, 1)
    inv = jnp.where(cnt > 0, 1.0 / cnt, 0.0)
    out[...] = s * inv


def _stage2(partials, counts):
    blk = 2048
    return pl.pallas_call(
        _stage2_body,
        grid=(SEG_PAD // blk,),
        in_specs=[
            pl.BlockSpec((NC, blk, D), lambda i: (0, i, 0)),
            pl.BlockSpec((NW, 2, blk), lambda i: (0, 0, i)),
        ],
        out_specs=pl.BlockSpec((blk, D), lambda i: (i, 0)),
        out_shape=jax.ShapeDtypeStruct((SEG_PAD, D), jnp.float32),
    )(partials, counts)


def _stage3_body(p, cnts, batch, bc, Wout, bout, Wg1, bg1, Wg2, bg2, out):
    s = p[0, :N, :] + p[1, :N, :]
    cnt = jnp.sum(cnts[:, 0, :N], axis=0).reshape(N, 1)
    inv = jnp.where(cnt > 0, 1.0 / cnt, 0.0)
    h = s * inv + bc[...]
    o = jnp.dot(h, Wout[...], preferred_element_type=jnp.float32) + bout[...]
    g1 = jnp.tanh(
        jnp.dot(o, Wg1[...], preferred_element_type=jnp.float32) + bg1[...])
    gate = jnp.dot(g1, Wg2[...], preferred_element_type=jnp.float32) + bg2[...]
    b = batch[...]
    gid = lax.broadcasted_iota(jnp.int32, (1, NUM_GRAPHS), 1)
    mask = b == gid
    maskf = mask.astype(jnp.float32)
    gmax = jnp.max(jnp.where(mask, gate, -1e30), axis=0, keepdims=True)
    grow = jnp.sum(maskf * gmax, axis=1, keepdims=True)
    e = jnp.exp(gate - grow)
    denom = jnp.sum(maskf * e, axis=0, keepdims=True)
    drow = jnp.sum(maskf * denom, axis=1, keepdims=True)
    alpha = e / (drow + 1e-16)
    w = maskf * alpha
    out[...] = lax.dot_general(
        w, o, dimension_numbers=(((0,), (0,)), ((), ())),
        preferred_element_type=jnp.float32)


def _stage3(partials, counts, batch2d, bc, Wout, bout, Wg1, bg1, Wg2, bg2):
    return pl.pallas_call(
        _stage3_body,
        out_shape=jax.ShapeDtypeStruct((NUM_GRAPHS, D), jnp.float32),
    )(partials, counts, batch2d, bc, Wout, bout, Wg1, bg1, Wg2, bg2)


def kernel(x, edge_index, batch, W1, b1, gamma1, beta1, Wc, bc, Wout, bout,
           Wg1, bg1, Wg2, bg2):
    node_idx = edge_index[0]
    he_idx = edge_index[1]
    g = gamma1 / jnp.sqrt(1.0 + EPS_BN)
    W1f = W1 * g[None, :]
    b1f = (b1 * g + beta1).reshape(1, D)

    counts = _sc_degrees(node_idx, he_idx)
    h2 = _stage1(x, W1f, b1f, Wc)
    p1 = _sc_propagate(h2, node_idx, he_idx)
    ef = _stage2(p1, counts)
    p2 = _sc_propagate(ef, he_idx, node_idx)
    return _stage3(p2, counts, batch.reshape(N, 1), bc.reshape(1, D),
                   Wout, bout.reshape(1, D), Wg1, bg1.reshape(1, D // 2),
                   Wg2, bg2.reshape(1, 1))


# speedup vs baseline: 26.5078x; 1.3830x over previous
import functools

import jax
import jax.numpy as jnp
from jax import lax
from jax.experimental import pallas as pl
from jax.experimental.pallas import tpu as pltpu
from jax.experimental.pallas import tpu_sc as plsc

N = 10000
E = 320000
NUM_SEG = 10000
D = 128
NUM_GRAPHS = 8
EPS_BN = 1e-5

NC = 2
NS = 16
NW = NC * NS
E_PER_W = E // NW
CHUNK = 40
PASSES = 2
E_PER_P = E_PER_W // PASSES
NCHUNK_P = E_PER_P // CHUNK
NCP_PAD = 128
IG = 16
GSTAGE = 1000
NBUF = 4
SEG_PAD = 10240
ROWS_PER_TILE = SEG_PAD // NS
ZROWS = 40
CPT = SEG_PAD // NS


def _sc_degrees_body(nidx_hbm, hidx_hbm, out_hbm,
                     locn, loch, ibufn, ibufh):
    c = lax.axis_index("c")
    s = lax.axis_index("s")
    wid = c * NS + s
    base = pl.multiple_of(wid * E_PER_W, 8)

    zero16 = jnp.zeros((16,), jnp.float32)
    one16 = jnp.ones((16,), jnp.float32)

    def zz(i, _):
        locn[pl.ds(i * 16, 16)] = zero16
        loch[pl.ds(i * 16, 16)] = zero16
        return 0
    lax.fori_loop(0, SEG_PAD // 16, zz, 0)

    pltpu.sync_copy(nidx_hbm.at[pl.ds(base, E_PER_W)], ibufn)
    pltpu.sync_copy(hidx_hbm.at[pl.ds(base, E_PER_W)], ibufh)

    def cnt(k, _):
        plsc.addupdate_scatter(locn, [ibufn[pl.ds(k * 16, 16)]], one16)
        plsc.addupdate_scatter(loch, [ibufh[pl.ds(k * 16, 16)]], one16)
        return 0
    lax.fori_loop(0, E_PER_W // 16, cnt, 0)

    pltpu.sync_copy(locn, out_hbm.at[wid, 0])
    pltpu.sync_copy(loch, out_hbm.at[wid, 1])


def _sc_degrees(nidx, hidx):
    mesh = plsc.VectorSubcoreMesh(core_axis_name="c", subcore_axis_name="s")
    kern = functools.partial(
        pl.kernel,
        mesh=mesh,
        compiler_params=pltpu.CompilerParams(needs_layout_passes=False),
        out_type=jax.ShapeDtypeStruct((NW, 2, SEG_PAD), jnp.float32),
        scratch_types=[
            pltpu.VMEM((SEG_PAD,), jnp.float32),
            pltpu.VMEM((SEG_PAD,), jnp.float32),
            pltpu.VMEM((E_PER_W,), jnp.int32),
            pltpu.VMEM((E_PER_W,), jnp.int32),
        ],
    )(_sc_degrees_body)
    return kern(nidx, hidx)


def _sc_propagate_body(table, gidx_hbm, sidx_hbm, out_hbm,
                       acc, gbuf, sbuf, rows, zbuf, semg):
    c = lax.axis_index("c")
    s = lax.axis_index("s")
    wid = c * NS + s

    zero16 = jnp.zeros((16,), jnp.float32)

    def zrow(i, _):
        r = i // (D // 16)
        k = i % (D // 16)
        zbuf[r, pl.ds(k * 16, 16)] = zero16
        return 0
    lax.fori_loop(0, ZROWS * (D // 16), zrow, 0)

    def zcp(q, _):
        ro = pl.multiple_of(s * ROWS_PER_TILE + q * ZROWS, 8)
        pltpu.sync_copy(zbuf, acc.at[pl.ds(ro, ZROWS)])
        return 0
    lax.fori_loop(0, ROWS_PER_TILE // ZROWS, zcp, 0)
    plsc.subcore_barrier()

    def run_pass(p, _):
        def gcp(g, _):
            ro = pl.multiple_of(g * GSTAGE, 8)
            src = pl.multiple_of(wid * E_PER_W + p * E_PER_P + ro, 8)
            pltpu.sync_copy(gidx_hbm.at[pl.ds(src, GSTAGE)],
                            gbuf.at[pl.ds(ro, GSTAGE)])
            return 0
        lax.fori_loop(0, E_PER_P // GSTAGE, gcp, 0)

        def scp(g, _):
            ro = pl.multiple_of(g * IG, 8)
            pltpu.sync_copy(sidx_hbm.at[wid, p, pl.ds(ro, IG)],
                            sbuf.at[pl.ds(ro, IG)])
            return 0
        lax.fori_loop(0, NCP_PAD // IG, scp, 0)

        def rslice(j):
            off = pl.multiple_of((j % NBUF) * CHUNK, 8)
            return rows.at[pl.ds(off, CHUNK)]

        def gslice(j):
            off = pl.multiple_of(j * CHUNK, 8)
            return gbuf.at[pl.ds(off, CHUNK)]

        def step(t, _):
            @pl.when(t < NCHUNK_P)
            def _():
                pltpu.async_copy(table.at[gslice(t)], rslice(t), semg)

            @pl.when(t >= NBUF - 1)
            def _():
                j = t - (NBUF - 1)
                pltpu.make_async_copy(table.at[gslice(j)], rslice(j),
                                      semg).wait()
                pltpu.sync_copy(rslice(j), acc.at[sbuf.at[j]], add=True)
            return 0
        lax.fori_loop(0, NCHUNK_P + NBUF - 1, step, 0)
        return 0

    lax.fori_loop(0, PASSES, run_pass, 0)
    plsc.subcore_barrier()

    ro = pl.multiple_of(s * ROWS_PER_TILE, 8)
    pltpu.sync_copy(acc.at[pl.ds(ro, ROWS_PER_TILE)],
                    out_hbm.at[c, pl.ds(ro, ROWS_PER_TILE)])


def _sc_propagate(table, gidx, sidx):
    mesh = plsc.VectorSubcoreMesh(core_axis_name="c", subcore_axis_name="s")
    kern = functools.partial(
        pl.kernel,
        mesh=mesh,
        compiler_params=pltpu.CompilerParams(needs_layout_passes=False),
        out_type=jax.ShapeDtypeStruct((NC, SEG_PAD, D), jnp.float32),
        scratch_types=[
            pltpu.VMEM_SHARED((SEG_PAD, D), jnp.float32),
            pltpu.VMEM((E_PER_P,), jnp.int32),
            pltpu.VMEM((NCP_PAD, CHUNK), jnp.int32),
            pltpu.VMEM((NBUF * CHUNK, D), jnp.float32),
            pltpu.VMEM((ZROWS, D), jnp.float32),
            pltpu.SemaphoreType.DMA,
        ],
    )(_sc_propagate_body)
    sidx_p = jnp.pad(sidx.reshape(NW, PASSES, NCHUNK_P, CHUNK),
                     ((0, 0), (0, 0), (0, NCP_PAD - NCHUNK_P), (0, 0)))
    return kern(table, gidx, sidx_p)


def _stage1_body(x, W1, b1s, Wc, out):
    h = jnp.dot(x[...], W1[...], preferred_element_type=jnp.float32)
    h = jax.nn.relu(h + b1s[...])
    out[...] = jnp.dot(h, Wc[...], preferred_element_type=jnp.float32)


def _stage1(x, W1, b1s, Wc):
    blk = 2000
    return pl.pallas_call(
        _stage1_body,
        grid=(N // blk,),
        in_specs=[
            pl.BlockSpec((blk, D), lambda i: (i, 0)),
            pl.BlockSpec((D, D), lambda i: (0, 0)),
            pl.BlockSpec((1, D), lambda i: (0, 0)),
            pl.BlockSpec((D, D), lambda i: (0, 0)),
        ],
        out_specs=pl.BlockSpec((blk, D), lambda i: (i, 0)),
        out_shape=jax.ShapeDtypeStruct((N, D), jnp.float32),
    )(x, W1, b1s, Wc)


def _stage2_body(p, cnts, out):
    s = p[0] + p[1]
    cnt = jnp.sum(cnts[:, 1, :], axis=0).reshape(-1, 1)
    inv = jnp.where(cnt > 0, 1.0 / cnt, 0.0)
    out[...] = s * inv


def _stage2(partials, counts):
    blk = 2048
    return pl.pallas_call(
        _stage2_body,
        grid=(SEG_PAD // blk,),
        in_specs=[
            pl.BlockSpec((NC, blk, D), lambda i: (0, i, 0)),
            pl.BlockSpec((NW, 2, blk), lambda i: (0, 0, i)),
        ],
        out_specs=pl.BlockSpec((blk, D), lambda i: (i, 0)),
        out_shape=jax.ShapeDtypeStruct((SEG_PAD, D), jnp.float32),
    )(partials, counts)


def _stage3_body(p, cnts, batch, bc, Wout, bout, Wg1, bg1, Wg2, bg2, out):
    s = p[0, :N, :] + p[1, :N, :]
    cnt = jnp.sum(cnts[:, 0, :N], axis=0).reshape(N, 1)
    inv = jnp.where(cnt > 0, 1.0 / cnt, 0.0)
    h = s * inv + bc[...]
    o = jnp.dot(h, Wout[...], preferred_element_type=jnp.float32) + bout[...]
    g1 = jnp.tanh(
        jnp.dot(o, Wg1[...], preferred_element_type=jnp.float32) + bg1[...])
    gate = jnp.dot(g1, Wg2[...], preferred_element_type=jnp.float32) + bg2[...]
    b = batch[...]
    gid = lax.broadcasted_iota(jnp.int32, (1, NUM_GRAPHS), 1)
    mask = b == gid
    maskf = mask.astype(jnp.float32)
    gmax = jnp.max(jnp.where(mask, gate, -1e30), axis=0, keepdims=True)
    grow = jnp.sum(maskf * gmax, axis=1, keepdims=True)
    e = jnp.exp(gate - grow)
    denom = jnp.sum(maskf * e, axis=0, keepdims=True)
    drow = jnp.sum(maskf * denom, axis=1, keepdims=True)
    alpha = e / (drow + 1e-16)
    w = maskf * alpha
    out[...] = lax.dot_general(
        w, o, dimension_numbers=(((0,), (0,)), ((), ())),
        preferred_element_type=jnp.float32)


def _stage3(partials, counts, batch2d, bc, Wout, bout, Wg1, bg1, Wg2, bg2):
    return pl.pallas_call(
        _stage3_body,
        out_shape=jax.ShapeDtypeStruct((NUM_GRAPHS, D), jnp.float32),
    )(partials, counts, batch2d, bc, Wout, bout, Wg1, bg1, Wg2, bg2)


def kernel(x, edge_index, batch, W1, b1, gamma1, beta1, Wc, bc, Wout, bout,
           Wg1, bg1, Wg2, bg2):
    node_idx = edge_index[0]
    he_idx = edge_index[1]
    g = gamma1 / jnp.sqrt(1.0 + EPS_BN)
    W1f = W1 * g[None, :]
    b1f = (b1 * g + beta1).reshape(1, D)

    counts = _sc_degrees(node_idx, he_idx)
    h2 = _stage1(x, W1f, b1f, Wc)
    p1 = _sc_propagate(h2, node_idx, he_idx)
    ef = _stage2(p1, counts)
    p2 = _sc_propagate(ef, he_idx, node_idx)
    return _stage3(p2, counts, batch.reshape(N, 1), bc.reshape(1, D),
                   Wout, bout.reshape(1, D), Wg1, bg1.reshape(1, D // 2),
                   Wg2, bg2.reshape(1, 1))


# speedup vs baseline: 26.9276x; 1.0158x over previous
import functools

import jax
import jax.numpy as jnp
from jax import lax
from jax.experimental import pallas as pl
from jax.experimental.pallas import tpu as pltpu
from jax.experimental.pallas import tpu_sc as plsc

N = 10000
E = 320000
NUM_SEG = 10000
D = 128
NUM_GRAPHS = 8
EPS_BN = 1e-5

NC = 2
NS = 16
NW = NC * NS
E_PER_W = E // NW
CHUNK = 40
PASSES = 2
E_PER_P = E_PER_W // PASSES
NCHUNK_P = E_PER_P // CHUNK
NCP_PAD = 128
IG = 16
GSTAGE = 1000
NBUF = 4
SEG_PAD = 10240
ROWS_PER_TILE = SEG_PAD // NS
ZROWS = 40
CPT = SEG_PAD // NS


def _sc_degrees_body(nidx_hbm, hidx_hbm, out_hbm,
                     locn, loch, ibufn, ibufh):
    c = lax.axis_index("c")
    s = lax.axis_index("s")
    wid = c * NS + s
    base = pl.multiple_of(wid * E_PER_W, 8)

    zero16 = jnp.zeros((16,), jnp.float32)
    one16 = jnp.ones((16,), jnp.float32)

    def zz(i, _):
        locn[pl.ds(i * 16, 16)] = zero16
        loch[pl.ds(i * 16, 16)] = zero16
        return 0
    lax.fori_loop(0, SEG_PAD // 16, zz, 0)

    pltpu.sync_copy(nidx_hbm.at[pl.ds(base, E_PER_W)], ibufn)
    pltpu.sync_copy(hidx_hbm.at[pl.ds(base, E_PER_W)], ibufh)

    def cnt(k, _):
        plsc.addupdate_scatter(locn, [ibufn[pl.ds(k * 16, 16)]], one16)
        plsc.addupdate_scatter(loch, [ibufh[pl.ds(k * 16, 16)]], one16)
        return 0
    lax.fori_loop(0, E_PER_W // 16, cnt, 0)

    pltpu.sync_copy(locn, out_hbm.at[wid, 0])
    pltpu.sync_copy(loch, out_hbm.at[wid, 1])


def _sc_degrees(nidx, hidx):
    mesh = plsc.VectorSubcoreMesh(core_axis_name="c", subcore_axis_name="s")
    kern = functools.partial(
        pl.kernel,
        mesh=mesh,
        compiler_params=pltpu.CompilerParams(needs_layout_passes=False),
        out_type=jax.ShapeDtypeStruct((NW, 2, SEG_PAD), jnp.float32),
        scratch_types=[
            pltpu.VMEM((SEG_PAD,), jnp.float32),
            pltpu.VMEM((SEG_PAD,), jnp.float32),
            pltpu.VMEM((E_PER_W,), jnp.int32),
            pltpu.VMEM((E_PER_W,), jnp.int32),
        ],
    )(_sc_degrees_body)
    return kern(nidx, hidx)


def _sc_propagate_body(table, gidx_hbm, sidx_hbm, out_hbm,
                       acc, gbuf, sbuf, rows, zbuf, semg, semz):
    c = lax.axis_index("c")
    s = lax.axis_index("s")
    wid = c * NS + s

    zero16 = jnp.zeros((16,), jnp.float32)

    def zrow(i, _):
        r = i // (D // 16)
        k = i % (D // 16)
        zbuf[r, pl.ds(k * 16, 16)] = zero16
        return 0
    lax.fori_loop(0, ZROWS * (D // 16), zrow, 0)

    def zcp(q, _):
        ro = pl.multiple_of(s * ROWS_PER_TILE + q * ZROWS, 8)
        pltpu.async_copy(zbuf, acc.at[pl.ds(ro, ZROWS)], semz)
        return 0
    lax.fori_loop(0, ROWS_PER_TILE // ZROWS, zcp, 0)

    def run_pass(p, _):
        def gcp(g, _):
            ro = pl.multiple_of(g * GSTAGE, 8)
            src = pl.multiple_of(wid * E_PER_W + p * E_PER_P + ro, 8)
            pltpu.sync_copy(gidx_hbm.at[pl.ds(src, GSTAGE)],
                            gbuf.at[pl.ds(ro, GSTAGE)])
            return 0
        lax.fori_loop(0, E_PER_P // GSTAGE, gcp, 0)

        def scp(g, _):
            ro = pl.multiple_of(g * IG, 8)
            pltpu.sync_copy(sidx_hbm.at[wid, p, pl.ds(ro, IG)],
                            sbuf.at[pl.ds(ro, IG)])
            return 0
        lax.fori_loop(0, NCP_PAD // IG, scp, 0)

        @pl.when(p == 0)
        def _():
            def zw(q, _):
                ro = pl.multiple_of(s * ROWS_PER_TILE + q * ZROWS, 8)
                pltpu.make_async_copy(zbuf, acc.at[pl.ds(ro, ZROWS)],
                                      semz).wait()
                return 0
            lax.fori_loop(0, ROWS_PER_TILE // ZROWS, zw, 0)
            plsc.subcore_barrier()

        def rslice(j):
            off = pl.multiple_of((j % NBUF) * CHUNK, 8)
            return rows.at[pl.ds(off, CHUNK)]

        def gslice(j):
            off = pl.multiple_of(j * CHUNK, 8)
            return gbuf.at[pl.ds(off, CHUNK)]

        def step(t, _):
            @pl.when(t < NCHUNK_P)
            def _():
                pltpu.async_copy(table.at[gslice(t)], rslice(t), semg)

            @pl.when(t >= NBUF - 1)
            def _():
                j = t - (NBUF - 1)
                pltpu.make_async_copy(table.at[gslice(j)], rslice(j),
                                      semg).wait()
                pltpu.sync_copy(rslice(j), acc.at[sbuf.at[j]], add=True)
            return 0
        lax.fori_loop(0, NCHUNK_P + NBUF - 1, step, 0)
        return 0

    lax.fori_loop(0, PASSES, run_pass, 0)
    plsc.subcore_barrier()

    ro = pl.multiple_of(s * ROWS_PER_TILE, 8)
    pltpu.sync_copy(acc.at[pl.ds(ro, ROWS_PER_TILE)],
                    out_hbm.at[c, pl.ds(ro, ROWS_PER_TILE)])


def _sc_propagate(table, gidx, sidx):
    mesh = plsc.VectorSubcoreMesh(core_axis_name="c", subcore_axis_name="s")
    kern = functools.partial(
        pl.kernel,
        mesh=mesh,
        compiler_params=pltpu.CompilerParams(needs_layout_passes=False),
        out_type=jax.ShapeDtypeStruct((NC, SEG_PAD, D), jnp.float32),
        scratch_types=[
            pltpu.VMEM_SHARED((SEG_PAD, D), jnp.float32),
            pltpu.VMEM((E_PER_P,), jnp.int32),
            pltpu.VMEM((NCP_PAD, CHUNK), jnp.int32),
            pltpu.VMEM((NBUF * CHUNK, D), jnp.float32),
            pltpu.VMEM((ZROWS, D), jnp.float32),
            pltpu.SemaphoreType.DMA,
            pltpu.SemaphoreType.DMA,
        ],
    )(_sc_propagate_body)
    sidx_p = jnp.pad(sidx.reshape(NW, PASSES, NCHUNK_P, CHUNK),
                     ((0, 0), (0, 0), (0, NCP_PAD - NCHUNK_P), (0, 0)))
    return kern(table, gidx, sidx_p)


def _stage1_body(x, W1, b1s, Wc, out):
    h = jnp.dot(x[...], W1[...], preferred_element_type=jnp.float32)
    h = jax.nn.relu(h + b1s[...])
    out[...] = jnp.dot(h, Wc[...], preferred_element_type=jnp.float32)


def _stage1(x, W1, b1s, Wc):
    blk = 2000
    return pl.pallas_call(
        _stage1_body,
        grid=(N // blk,),
        in_specs=[
            pl.BlockSpec((blk, D), lambda i: (i, 0)),
            pl.BlockSpec((D, D), lambda i: (0, 0)),
            pl.BlockSpec((1, D), lambda i: (0, 0)),
            pl.BlockSpec((D, D), lambda i: (0, 0)),
        ],
        out_specs=pl.BlockSpec((blk, D), lambda i: (i, 0)),
        out_shape=jax.ShapeDtypeStruct((N, D), jnp.float32),
    )(x, W1, b1s, Wc)


def _stage2_body(p, cnts, out):
    s = p[0] + p[1]
    cnt = jnp.sum(cnts[:, 1, :], axis=0).reshape(-1, 1)
    inv = jnp.where(cnt > 0, 1.0 / cnt, 0.0)
    out[...] = s * inv


def _stage2(partials, counts):
    blk = 2048
    return pl.pallas_call(
        _stage2_body,
        grid=(SEG_PAD // blk,),
        in_specs=[
            pl.BlockSpec((NC, blk, D), lambda i: (0, i, 0)),
            pl.BlockSpec((NW, 2, blk), lambda i: (0, 0, i)),
        ],
        out_specs=pl.BlockSpec((blk, D), lambda i: (i, 0)),
        out_shape=jax.ShapeDtypeStruct((SEG_PAD, D), jnp.float32),
    )(partials, counts)


def _stage3_body(p, cnts, batch, bc, Wout, bout, Wg1, bg1, Wg2, bg2, out):
    s = p[0, :N, :] + p[1, :N, :]
    cnt = jnp.sum(cnts[:, 0, :N], axis=0).reshape(N, 1)
    inv = jnp.where(cnt > 0, 1.0 / cnt, 0.0)
    h = s * inv + bc[...]
    o = jnp.dot(h, Wout[...], preferred_element_type=jnp.float32) + bout[...]
    g1 = jnp.tanh(
        jnp.dot(o, Wg1[...], preferred_element_type=jnp.float32) + bg1[...])
    gate = jnp.dot(g1, Wg2[...], preferred_element_type=jnp.float32) + bg2[...]
    b = batch[...]
    gid = lax.broadcasted_iota(jnp.int32, (1, NUM_GRAPHS), 1)
    mask = b == gid
    maskf = mask.astype(jnp.float32)
    gmax = jnp.max(jnp.where(mask, gate, -1e30), axis=0, keepdims=True)
    grow = jnp.sum(maskf * gmax, axis=1, keepdims=True)
    e = jnp.exp(gate - grow)
    denom = jnp.sum(maskf * e, axis=0, keepdims=True)
    drow = jnp.sum(maskf * denom, axis=1, keepdims=True)
    alpha = e / (drow + 1e-16)
    w = maskf * alpha
    out[...] = lax.dot_general(
        w, o, dimension_numbers=(((0,), (0,)), ((), ())),
        preferred_element_type=jnp.float32)


def _stage3(partials, counts, batch2d, bc, Wout, bout, Wg1, bg1, Wg2, bg2):
    return pl.pallas_call(
        _stage3_body,
        out_shape=jax.ShapeDtypeStruct((NUM_GRAPHS, D), jnp.float32),
    )(partials, counts, batch2d, bc, Wout, bout, Wg1, bg1, Wg2, bg2)


def kernel(x, edge_index, batch, W1, b1, gamma1, beta1, Wc, bc, Wout, bout,
           Wg1, bg1, Wg2, bg2):
    node_idx = edge_index[0]
    he_idx = edge_index[1]
    g = gamma1 / jnp.sqrt(1.0 + EPS_BN)
    W1f = W1 * g[None, :]
    b1f = (b1 * g + beta1).reshape(1, D)

    counts = _sc_degrees(node_idx, he_idx)
    h2 = _stage1(x, W1f, b1f, Wc)
    p1 = _sc_propagate(h2, node_idx, he_idx)
    ef = _stage2(p1, counts)
    p2 = _sc_propagate(ef, he_idx, node_idx)
    return _stage3(p2, counts, batch.reshape(N, 1), bc.reshape(1, D),
                   Wout, bout.reshape(1, D), Wg1, bg1.reshape(1, D // 2),
                   Wg2, bg2.reshape(1, 1))


# speedup vs baseline: 28.3912x; 1.0544x over previous
import functools

import jax
import jax.numpy as jnp
from jax import lax
from jax.experimental import pallas as pl
from jax.experimental.pallas import tpu as pltpu
from jax.experimental.pallas import tpu_sc as plsc

N = 10000
E = 320000
NUM_SEG = 10000
D = 128
NUM_GRAPHS = 8
EPS_BN = 1e-5

NC = 2
NS = 16
NW = NC * NS
E_PER_W = E // NW
CHUNK = 40
PASSES = 2
E_PER_P = E_PER_W // PASSES
NCHUNK_P = E_PER_P // CHUNK
NCP_PAD = 128
IG = 128
GSTAGE = 1000
NBUF = 4
SEG_PAD = 10240
ROWS_PER_TILE = SEG_PAD // NS
ZROWS = 40
CPT = SEG_PAD // NS


def _sc_degrees_body(nidx_hbm, hidx_hbm, out_hbm,
                     locn, loch, ibufn, ibufh):
    c = lax.axis_index("c")
    s = lax.axis_index("s")
    wid = c * NS + s
    base = pl.multiple_of(wid * E_PER_W, 8)

    zero16 = jnp.zeros((16,), jnp.float32)
    one16 = jnp.ones((16,), jnp.float32)

    def zz(i, _):
        locn[pl.ds(i * 16, 16)] = zero16
        loch[pl.ds(i * 16, 16)] = zero16
        return 0
    lax.fori_loop(0, SEG_PAD // 16, zz, 0)

    pltpu.sync_copy(nidx_hbm.at[pl.ds(base, E_PER_W)], ibufn)
    pltpu.sync_copy(hidx_hbm.at[pl.ds(base, E_PER_W)], ibufh)

    def cnt(k, _):
        plsc.addupdate_scatter(locn, [ibufn[pl.ds(k * 16, 16)]], one16)
        plsc.addupdate_scatter(loch, [ibufh[pl.ds(k * 16, 16)]], one16)
        return 0
    lax.fori_loop(0, E_PER_W // 16, cnt, 0)

    pltpu.sync_copy(locn, out_hbm.at[wid, 0])
    pltpu.sync_copy(loch, out_hbm.at[wid, 1])


def _sc_degrees(nidx, hidx):
    mesh = plsc.VectorSubcoreMesh(core_axis_name="c", subcore_axis_name="s")
    kern = functools.partial(
        pl.kernel,
        mesh=mesh,
        compiler_params=pltpu.CompilerParams(needs_layout_passes=False),
        out_type=jax.ShapeDtypeStruct((NW, 2, SEG_PAD), jnp.float32),
        scratch_types=[
            pltpu.VMEM((SEG_PAD,), jnp.float32),
            pltpu.VMEM((SEG_PAD,), jnp.float32),
            pltpu.VMEM((E_PER_W,), jnp.int32),
            pltpu.VMEM((E_PER_W,), jnp.int32),
        ],
    )(_sc_degrees_body)
    return kern(nidx, hidx)


def _sc_propagate_body(table, gidx_hbm, sidx_hbm, out_hbm,
                       acc, gbuf, sbuf, rows, zbuf, semg, semz):
    c = lax.axis_index("c")
    s = lax.axis_index("s")
    wid = c * NS + s

    zero16 = jnp.zeros((16,), jnp.float32)

    def zrow(i, _):
        r = i // (D // 16)
        k = i % (D // 16)
        zbuf[r, pl.ds(k * 16, 16)] = zero16
        return 0
    lax.fori_loop(0, ZROWS * (D // 16), zrow, 0)

    def zcp(q, _):
        ro = pl.multiple_of(s * ROWS_PER_TILE + q * ZROWS, 8)
        pltpu.async_copy(zbuf, acc.at[pl.ds(ro, ZROWS)], semz)
        return 0
    lax.fori_loop(0, ROWS_PER_TILE // ZROWS, zcp, 0)

    def run_pass(p, _):
        def gcp(g, _):
            ro = pl.multiple_of(g * GSTAGE, 8)
            src = pl.multiple_of(wid * E_PER_W + p * E_PER_P + ro, 8)
            pltpu.sync_copy(gidx_hbm.at[pl.ds(src, GSTAGE)],
                            gbuf.at[pl.ds(ro, GSTAGE)])
            return 0
        lax.fori_loop(0, E_PER_P // GSTAGE, gcp, 0)

        def scp(g, _):
            ro = pl.multiple_of(g * IG, 8)
            pltpu.sync_copy(sidx_hbm.at[wid, p, pl.ds(ro, IG)],
                            sbuf.at[pl.ds(ro, IG)])
            return 0
        lax.fori_loop(0, NCP_PAD // IG, scp, 0)

        @pl.when(p == 0)
        def _():
            def zw(q, _):
                ro = pl.multiple_of(s * ROWS_PER_TILE + q * ZROWS, 8)
                pltpu.make_async_copy(zbuf, acc.at[pl.ds(ro, ZROWS)],
                                      semz).wait()
                return 0
            lax.fori_loop(0, ROWS_PER_TILE // ZROWS, zw, 0)
            plsc.subcore_barrier()

        def rslice(j):
            off = pl.multiple_of((j % NBUF) * CHUNK, 8)
            return rows.at[pl.ds(off, CHUNK)]

        def gslice(j):
            off = pl.multiple_of(j * CHUNK, 8)
            return gbuf.at[pl.ds(off, CHUNK)]

        def step(t, _):
            @pl.when(t < NCHUNK_P)
            def _():
                pltpu.async_copy(table.at[gslice(t)], rslice(t), semg)

            @pl.when(t >= NBUF - 1)
            def _():
                j = t - (NBUF - 1)
                pltpu.make_async_copy(table.at[gslice(j)], rslice(j),
                                      semg).wait()
                pltpu.sync_copy(rslice(j), acc.at[sbuf.at[j]], add=True)
            return 0
        lax.fori_loop(0, NCHUNK_P + NBUF - 1, step, 0)
        return 0

    lax.fori_loop(0, PASSES, run_pass, 0)
    plsc.subcore_barrier()

    ro = pl.multiple_of(s * ROWS_PER_TILE, 8)
    pltpu.sync_copy(acc.at[pl.ds(ro, ROWS_PER_TILE)],
                    out_hbm.at[c, pl.ds(ro, ROWS_PER_TILE)])


def _sc_propagate(table, gidx, sidx):
    mesh = plsc.VectorSubcoreMesh(core_axis_name="c", subcore_axis_name="s")
    kern = functools.partial(
        pl.kernel,
        mesh=mesh,
        compiler_params=pltpu.CompilerParams(needs_layout_passes=False),
        out_type=jax.ShapeDtypeStruct((NC, SEG_PAD, D), jnp.float32),
        scratch_types=[
            pltpu.VMEM_SHARED((SEG_PAD, D), jnp.float32),
            pltpu.VMEM((E_PER_P,), jnp.int32),
            pltpu.VMEM((NCP_PAD, CHUNK), jnp.int32),
            pltpu.VMEM((NBUF * CHUNK, D), jnp.float32),
            pltpu.VMEM((ZROWS, D), jnp.float32),
            pltpu.SemaphoreType.DMA,
            pltpu.SemaphoreType.DMA,
        ],
    )(_sc_propagate_body)
    sidx_p = jnp.pad(sidx.reshape(NW, PASSES, NCHUNK_P, CHUNK),
                     ((0, 0), (0, 0), (0, NCP_PAD - NCHUNK_P), (0, 0)))
    return kern(table, gidx, sidx_p)


def _stage1_body(x, W1, b1s, Wc, out):
    h = jnp.dot(x[...], W1[...], preferred_element_type=jnp.float32)
    h = jax.nn.relu(h + b1s[...])
    out[...] = jnp.dot(h, Wc[...], preferred_element_type=jnp.float32)


def _stage1(x, W1, b1s, Wc):
    blk = 2000
    return pl.pallas_call(
        _stage1_body,
        grid=(N // blk,),
        in_specs=[
            pl.BlockSpec((blk, D), lambda i: (i, 0)),
            pl.BlockSpec((D, D), lambda i: (0, 0)),
            pl.BlockSpec((1, D), lambda i: (0, 0)),
            pl.BlockSpec((D, D), lambda i: (0, 0)),
        ],
        out_specs=pl.BlockSpec((blk, D), lambda i: (i, 0)),
        out_shape=jax.ShapeDtypeStruct((N, D), jnp.float32),
    )(x, W1, b1s, Wc)


def _stage2_body(p, cnts, out):
    s = p[0] + p[1]
    cnt = jnp.sum(cnts[:, 1, :], axis=0).reshape(-1, 1)
    inv = jnp.where(cnt > 0, 1.0 / cnt, 0.0)
    out[...] = s * inv


def _stage2(partials, counts):
    blk = 2048
    return pl.pallas_call(
        _stage2_body,
        grid=(SEG_PAD // blk,),
        in_specs=[
            pl.BlockSpec((NC, blk, D), lambda i: (0, i, 0)),
            pl.BlockSpec((NW, 2, blk), lambda i: (0, 0, i)),
        ],
        out_specs=pl.BlockSpec((blk, D), lambda i: (i, 0)),
        out_shape=jax.ShapeDtypeStruct((SEG_PAD, D), jnp.float32),
    )(partials, counts)


def _stage3_body(p, cnts, batch, bc, Wout, bout, Wg1, bg1, Wg2, bg2, out):
    s = p[0, :N, :] + p[1, :N, :]
    cnt = jnp.sum(cnts[:, 0, :N], axis=0).reshape(N, 1)
    inv = jnp.where(cnt > 0, 1.0 / cnt, 0.0)
    h = s * inv + bc[...]
    o = jnp.dot(h, Wout[...], preferred_element_type=jnp.float32) + bout[...]
    g1 = jnp.tanh(
        jnp.dot(o, Wg1[...], preferred_element_type=jnp.float32) + bg1[...])
    gate = jnp.dot(g1, Wg2[...], preferred_element_type=jnp.float32) + bg2[...]
    b = batch[...]
    gid = lax.broadcasted_iota(jnp.int32, (1, NUM_GRAPHS), 1)
    mask = b == gid
    maskf = mask.astype(jnp.float32)
    gmax = jnp.max(jnp.where(mask, gate, -1e30), axis=0, keepdims=True)
    grow = jnp.sum(maskf * gmax, axis=1, keepdims=True)
    e = jnp.exp(gate - grow)
    denom = jnp.sum(maskf * e, axis=0, keepdims=True)
    drow = jnp.sum(maskf * denom, axis=1, keepdims=True)
    alpha = e / (drow + 1e-16)
    w = maskf * alpha
    out[...] = lax.dot_general(
        w, o, dimension_numbers=(((0,), (0,)), ((), ())),
        preferred_element_type=jnp.float32)


def _stage3(partials, counts, batch2d, bc, Wout, bout, Wg1, bg1, Wg2, bg2):
    return pl.pallas_call(
        _stage3_body,
        out_shape=jax.ShapeDtypeStruct((NUM_GRAPHS, D), jnp.float32),
    )(partials, counts, batch2d, bc, Wout, bout, Wg1, bg1, Wg2, bg2)


def kernel(x, edge_index, batch, W1, b1, gamma1, beta1, Wc, bc, Wout, bout,
           Wg1, bg1, Wg2, bg2):
    node_idx = edge_index[0]
    he_idx = edge_index[1]
    g = gamma1 / jnp.sqrt(1.0 + EPS_BN)
    W1f = W1 * g[None, :]
    b1f = (b1 * g + beta1).reshape(1, D)

    counts = _sc_degrees(node_idx, he_idx)
    h2 = _stage1(x, W1f, b1f, Wc)
    p1 = _sc_propagate(h2, node_idx, he_idx)
    ef = _stage2(p1, counts)
    p2 = _sc_propagate(ef, he_idx, node_idx)
    return _stage3(p2, counts, batch.reshape(N, 1), bc.reshape(1, D),
                   Wout, bout.reshape(1, D), Wg1, bg1.reshape(1, D // 2),
                   Wg2, bg2.reshape(1, 1))


# speedup vs baseline: 29.2004x; 1.0285x over previous
import functools

import jax
import jax.numpy as jnp
from jax import lax
from jax.experimental import pallas as pl
from jax.experimental.pallas import tpu as pltpu
from jax.experimental.pallas import tpu_sc as plsc

N = 10000
E = 320000
NUM_SEG = 10000
D = 128
NUM_GRAPHS = 8
EPS_BN = 1e-5

NC = 2
NS = 16
NW = NC * NS
E_PER_W = E // NW
CHUNK = 40
PASSES = 2
E_PER_P = E_PER_W // PASSES
NCHUNK_P = E_PER_P // CHUNK
NCP_PAD = 128
IG = 128
GSTAGE = 5000
NBUF = 4
SEG_PAD = 10240
ROWS_PER_TILE = SEG_PAD // NS
ZROWS = 40
CPT = SEG_PAD // NS


def _sc_degrees_body(nidx_hbm, hidx_hbm, out_hbm,
                     locn, loch, ibufn, ibufh):
    c = lax.axis_index("c")
    s = lax.axis_index("s")
    wid = c * NS + s
    base = pl.multiple_of(wid * E_PER_W, 8)

    zero16 = jnp.zeros((16,), jnp.float32)
    one16 = jnp.ones((16,), jnp.float32)

    def zz(i, _):
        locn[pl.ds(i * 16, 16)] = zero16
        loch[pl.ds(i * 16, 16)] = zero16
        return 0
    lax.fori_loop(0, SEG_PAD // 16, zz, 0)

    pltpu.sync_copy(nidx_hbm.at[pl.ds(base, E_PER_W)], ibufn)
    pltpu.sync_copy(hidx_hbm.at[pl.ds(base, E_PER_W)], ibufh)

    def cnt(k, _):
        plsc.addupdate_scatter(locn, [ibufn[pl.ds(k * 16, 16)]], one16)
        plsc.addupdate_scatter(loch, [ibufh[pl.ds(k * 16, 16)]], one16)
        return 0
    lax.fori_loop(0, E_PER_W // 16, cnt, 0)

    pltpu.sync_copy(locn, out_hbm.at[wid, 0])
    pltpu.sync_copy(loch, out_hbm.at[wid, 1])


def _sc_degrees(nidx, hidx):
    mesh = plsc.VectorSubcoreMesh(core_axis_name="c", subcore_axis_name="s")
    kern = functools.partial(
        pl.kernel,
        mesh=mesh,
        compiler_params=pltpu.CompilerParams(needs_layout_passes=False),
        out_type=jax.ShapeDtypeStruct((NW, 2, SEG_PAD), jnp.float32),
        scratch_types=[
            pltpu.VMEM((SEG_PAD,), jnp.float32),
            pltpu.VMEM((SEG_PAD,), jnp.float32),
            pltpu.VMEM((E_PER_W,), jnp.int32),
            pltpu.VMEM((E_PER_W,), jnp.int32),
        ],
    )(_sc_degrees_body)
    return kern(nidx, hidx)


def _sc_propagate_body(table, gidx_hbm, sidx_hbm, out_hbm,
                       acc, gbuf, sbuf, rows, zbuf, semg, semz):
    c = lax.axis_index("c")
    s = lax.axis_index("s")
    wid = c * NS + s

    zero16 = jnp.zeros((16,), jnp.float32)

    def zrow(i, _):
        r = i // (D // 16)
        k = i % (D // 16)
        zbuf[r, pl.ds(k * 16, 16)] = zero16
        return 0
    lax.fori_loop(0, ZROWS * (D // 16), zrow, 0)

    def zcp(q, _):
        ro = pl.multiple_of(s * ROWS_PER_TILE + q * ZROWS, 8)
        pltpu.async_copy(zbuf, acc.at[pl.ds(ro, ZROWS)], semz)
        return 0
    lax.fori_loop(0, ROWS_PER_TILE // ZROWS, zcp, 0)

    def run_pass(p, _):
        def gcp(g, _):
            ro = pl.multiple_of(g * GSTAGE, 8)
            src = pl.multiple_of(wid * E_PER_W + p * E_PER_P + ro, 8)
            pltpu.sync_copy(gidx_hbm.at[pl.ds(src, GSTAGE)],
                            gbuf.at[pl.ds(ro, GSTAGE)])
            return 0
        lax.fori_loop(0, E_PER_P // GSTAGE, gcp, 0)

        def scp(g, _):
            ro = pl.multiple_of(g * IG, 8)
            pltpu.sync_copy(sidx_hbm.at[wid, p, pl.ds(ro, IG)],
                            sbuf.at[pl.ds(ro, IG)])
            return 0
        lax.fori_loop(0, NCP_PAD // IG, scp, 0)

        @pl.when(p == 0)
        def _():
            def zw(q, _):
                ro = pl.multiple_of(s * ROWS_PER_TILE + q * ZROWS, 8)
                pltpu.make_async_copy(zbuf, acc.at[pl.ds(ro, ZROWS)],
                                      semz).wait()
                return 0
            lax.fori_loop(0, ROWS_PER_TILE // ZROWS, zw, 0)
            plsc.subcore_barrier()

        def rslice(j):
            off = pl.multiple_of((j % NBUF) * CHUNK, 8)
            return rows.at[pl.ds(off, CHUNK)]

        def gslice(j):
            off = pl.multiple_of(j * CHUNK, 8)
            return gbuf.at[pl.ds(off, CHUNK)]

        def step(t, _):
            @pl.when(t < NCHUNK_P)
            def _():
                pltpu.async_copy(table.at[gslice(t)], rslice(t), semg)

            @pl.when(t >= NBUF - 1)
            def _():
                j = t - (NBUF - 1)
                pltpu.make_async_copy(table.at[gslice(j)], rslice(j),
                                      semg).wait()
                pltpu.sync_copy(rslice(j), acc.at[sbuf.at[j]], add=True)
            return 0
        lax.fori_loop(0, NCHUNK_P + NBUF - 1, step, 0)
        return 0

    lax.fori_loop(0, PASSES, run_pass, 0)
    plsc.subcore_barrier()

    ro = pl.multiple_of(s * ROWS_PER_TILE, 8)
    pltpu.sync_copy(acc.at[pl.ds(ro, ROWS_PER_TILE)],
                    out_hbm.at[c, pl.ds(ro, ROWS_PER_TILE)])


def _sc_propagate(table, gidx, sidx):
    mesh = plsc.VectorSubcoreMesh(core_axis_name="c", subcore_axis_name="s")
    kern = functools.partial(
        pl.kernel,
        mesh=mesh,
        compiler_params=pltpu.CompilerParams(needs_layout_passes=False),
        out_type=jax.ShapeDtypeStruct((NC, SEG_PAD, D), jnp.float32),
        scratch_types=[
            pltpu.VMEM_SHARED((SEG_PAD, D), jnp.float32),
            pltpu.VMEM((E_PER_P,), jnp.int32),
            pltpu.VMEM((NCP_PAD, CHUNK), jnp.int32),
            pltpu.VMEM((NBUF * CHUNK, D), jnp.float32),
            pltpu.VMEM((ZROWS, D), jnp.float32),
            pltpu.SemaphoreType.DMA,
            pltpu.SemaphoreType.DMA,
        ],
    )(_sc_propagate_body)
    sidx_p = jnp.pad(sidx.reshape(NW, PASSES, NCHUNK_P, CHUNK),
                     ((0, 0), (0, 0), (0, NCP_PAD - NCHUNK_P), (0, 0)))
    return kern(table, gidx, sidx_p)


def _stage1_body(x, W1, b1s, Wc, out):
    h = jnp.dot(x[...], W1[...], preferred_element_type=jnp.float32)
    h = jax.nn.relu(h + b1s[...])
    out[...] = jnp.dot(h, Wc[...], preferred_element_type=jnp.float32)


def _stage1(x, W1, b1s, Wc):
    blk = 2000
    return pl.pallas_call(
        _stage1_body,
        grid=(N // blk,),
        in_specs=[
            pl.BlockSpec((blk, D), lambda i: (i, 0)),
            pl.BlockSpec((D, D), lambda i: (0, 0)),
            pl.BlockSpec((1, D), lambda i: (0, 0)),
            pl.BlockSpec((D, D), lambda i: (0, 0)),
        ],
        out_specs=pl.BlockSpec((blk, D), lambda i: (i, 0)),
        out_shape=jax.ShapeDtypeStruct((N, D), jnp.float32),
    )(x, W1, b1s, Wc)


def _stage2_body(p, cnts, out):
    s = p[0] + p[1]
    cnt = jnp.sum(cnts[:, 1, :], axis=0).reshape(-1, 1)
    inv = jnp.where(cnt > 0, 1.0 / cnt, 0.0)
    out[...] = s * inv


def _stage2(partials, counts):
    blk = 2048
    return pl.pallas_call(
        _stage2_body,
        grid=(SEG_PAD // blk,),
        in_specs=[
            pl.BlockSpec((NC, blk, D), lambda i: (0, i, 0)),
            pl.BlockSpec((NW, 2, blk), lambda i: (0, 0, i)),
        ],
        out_specs=pl.BlockSpec((blk, D), lambda i: (i, 0)),
        out_shape=jax.ShapeDtypeStruct((SEG_PAD, D), jnp.float32),
    )(partials, counts)


def _stage3_body(p, cnts, batch, bc, Wout, bout, Wg1, bg1, Wg2, bg2, out):
    s = p[0, :N, :] + p[1, :N, :]
    cnt = jnp.sum(cnts[:, 0, :N], axis=0).reshape(N, 1)
    inv = jnp.where(cnt > 0, 1.0 / cnt, 0.0)
    h = s * inv + bc[...]
    o = jnp.dot(h, Wout[...], preferred_element_type=jnp.float32) + bout[...]
    g1 = jnp.tanh(
        jnp.dot(o, Wg1[...], preferred_element_type=jnp.float32) + bg1[...])
    gate = jnp.dot(g1, Wg2[...], preferred_element_type=jnp.float32) + bg2[...]
    b = batch[...]
    gid = lax.broadcasted_iota(jnp.int32, (1, NUM_GRAPHS), 1)
    mask = b == gid
    maskf = mask.astype(jnp.float32)
    gmax = jnp.max(jnp.where(mask, gate, -1e30), axis=0, keepdims=True)
    grow = jnp.sum(maskf * gmax, axis=1, keepdims=True)
    e = jnp.exp(gate - grow)
    denom = jnp.sum(maskf * e, axis=0, keepdims=True)
    drow = jnp.sum(maskf * denom, axis=1, keepdims=True)
    alpha = e / (drow + 1e-16)
    w = maskf * alpha
    out[...] = lax.dot_general(
        w, o, dimension_numbers=(((0,), (0,)), ((), ())),
        preferred_element_type=jnp.float32)


def _stage3(partials, counts, batch2d, bc, Wout, bout, Wg1, bg1, Wg2, bg2):
    return pl.pallas_call(
        _stage3_body,
        out_shape=jax.ShapeDtypeStruct((NUM_GRAPHS, D), jnp.float32),
    )(partials, counts, batch2d, bc, Wout, bout, Wg1, bg1, Wg2, bg2)


def kernel(x, edge_index, batch, W1, b1, gamma1, beta1, Wc, bc, Wout, bout,
           Wg1, bg1, Wg2, bg2):
    node_idx = edge_index[0]
    he_idx = edge_index[1]
    g = gamma1 / jnp.sqrt(1.0 + EPS_BN)
    W1f = W1 * g[None, :]
    b1f = (b1 * g + beta1).reshape(1, D)

    counts = _sc_degrees(node_idx, he_idx)
    h2 = _stage1(x, W1f, b1f, Wc)
    p1 = _sc_propagate(h2, node_idx, he_idx)
    ef = _stage2(p1, counts)
    p2 = _sc_propagate(ef, he_idx, node_idx)
    return _stage3(p2, counts, batch.reshape(N, 1), bc.reshape(1, D),
                   Wout, bout.reshape(1, D), Wg1, bg1.reshape(1, D // 2),
                   Wg2, bg2.reshape(1, 1))
